# baseline (speedup 1.0000x reference)
import functools

import jax
import jax.numpy as jnp
from jax import lax
from jax.experimental import pallas as pl
from jax.experimental.pallas import tpu as pltpu
from jax.experimental.pallas import tpu_sc as plsc

N = 320000
D = 128
L = 128
S = 10000

NC = 2
NS = 16
LN = 16
NW = NC * NS
Q = N // NW
SPAD = 10240
SEG = SPAD // NS
RCH = 400
NCHK = Q // RCH
GRP = Q // LN
NEG = -1e30

_mesh = plsc.VectorSubcoreMesh(
    core_axis_name="c", subcore_axis_name="s", num_cores=NC, num_subcores=NS)
_sc_params = pltpu.CompilerParams(needs_layout_passes=False)


_BR = 1600


def _dense_body(feat_ref, wat_ref, ba_ref, wbt_ref, bb_ref, wct_ref, bc_ref,
                f_ref, score_ref):
  x = feat_ref[...]
  ss = jnp.sum(x * x, axis=1, keepdims=True)
  f = x / jnp.maximum(jnp.sqrt(ss), 1e-12)
  a = jax.nn.sigmoid(
      jnp.dot(f, wat_ref[...], preferred_element_type=jnp.float32)
      + ba_ref[...])
  b = jnp.tanh(
      jnp.dot(f, wbt_ref[...], preferred_element_type=jnp.float32)
      + bb_ref[...])
  ab = a * b
  score = (jnp.dot(ab, wct_ref[...], preferred_element_type=jnp.float32)
           + bc_ref[0, 0])
  f_ref[...] = f
  score_ref[...] = score


def _dense(feature, WaT, ba2, WbT, bb2, WcT, bc2):
  return pl.pallas_call(
      _dense_body,
      grid=(N // _BR,),
      in_specs=[
          pl.BlockSpec((_BR, D), lambda i: (i, 0)),
          pl.BlockSpec((D, L), lambda i: (0, 0)),
          pl.BlockSpec((1, L), lambda i: (0, 0)),
          pl.BlockSpec((D, L), lambda i: (0, 0)),
          pl.BlockSpec((1, L), lambda i: (0, 0)),
          pl.BlockSpec((L, 1), lambda i: (0, 0)),
          pl.BlockSpec((1, 1), lambda i: (0, 0), memory_space=pltpu.SMEM),
      ],
      out_specs=[
          pl.BlockSpec((_BR, D), lambda i: (i, 0)),
          pl.BlockSpec((_BR, 1), lambda i: (i, 0)),
      ],
      out_shape=[
          jax.ShapeDtypeStruct((N, D), jnp.float32),
          jax.ShapeDtypeStruct((N, 1), jnp.float32),
      ],
  )(feature, WaT, ba2, WbT, bb2, WcT, bc2)


def _seg_scan(ids, v, lane, combine):
  for dlt in (1, 2, 4, 8):
    idx = jnp.maximum(lane - dlt, 0)
    vs = jnp.take_along_axis(v, idx, axis=0, mode="promise_in_bounds")
    is_ = jnp.take_along_axis(ids, idx, axis=0, mode="promise_in_bounds")
    ok = (lane >= dlt) & (is_ == ids)
    v = combine(v, vs, ok)
  return v


def _last_of_run(ids, lane):
  nxt = jnp.take_along_axis(ids, jnp.minimum(lane + 1, LN - 1), axis=0,
                            mode="promise_in_bounds")
  return (ids != nxt) | (lane == LN - 1)


def _segmax_body(score_hbm, batch_hbm, mpart_hbm, stat_v, sco_v, ids_v,
                 comb_v, stage_sh):
  c = lax.axis_index("c")
  s = lax.axis_index("s")
  w = c * NS + s
  neg = jnp.full((LN,), NEG, jnp.float32)

  def init(i, _):
    stat_v[pl.ds(i * LN, LN)] = neg
    return 0
  lax.fori_loop(0, SPAD // LN, init, 0)

  pltpu.sync_copy(score_hbm.at[pl.ds(w * Q, Q)], sco_v)
  pltpu.sync_copy(batch_hbm.at[pl.ds(w * Q, Q)], ids_v)
  lane = lax.iota(jnp.int32, LN)

  def body(g, _):
    ids = ids_v[pl.ds(g * LN, LN)]
    v = sco_v[pl.ds(g * LN, LN)]
    v = _seg_scan(ids, v, lane,
                  lambda a, b, ok: jnp.where(ok, jnp.maximum(a, b), a))
    last = _last_of_run(ids, lane)
    cur = plsc.load_gather(stat_v, [ids], mask=last)
    plsc.store_scatter(stat_v, [ids], jnp.maximum(cur, v), mask=last)
    return 0
  lax.fori_loop(0, GRP, body, 0)

  pltpu.sync_copy(stat_v, stage_sh.at[s])
  plsc.subcore_barrier()
  pltpu.sync_copy(stage_sh.at[:, pl.ds(s * SEG, SEG)], comb_v)

  def comb(i, _):
    acc = comb_v[0, pl.ds(i * LN, LN)]
    for r in range(1, NS):
      acc = jnp.maximum(acc, comb_v[r, pl.ds(i * LN, LN)])
    stat_v[pl.ds(i * LN, LN)] = acc
    return 0
  lax.fori_loop(0, SEG // LN, comb, 0)
  pltpu.sync_copy(stat_v.at[pl.ds(0, SEG)],
                  mpart_hbm.at[pl.ds(c * SPAD + s * SEG, SEG)])


_segmax = functools.partial(
    pl.kernel,
    out_type=jax.ShapeDtypeStruct((NC * SPAD,), jnp.float32),
    mesh=_mesh,
    compiler_params=_sc_params,
    scratch_types=[
        pltpu.VMEM((SPAD,), jnp.float32),
        pltpu.VMEM((Q,), jnp.float32),
        pltpu.VMEM((Q,), jnp.int32),
        pltpu.VMEM((NS, SEG), jnp.float32),
        pltpu.VMEM_SHARED((NS, SPAD), jnp.float32),
    ])(_segmax_body)


def _denom_body(score_hbm, batch_hbm, mpart_hbm, dpart_hbm, stat_v, sco_v,
                ids_v, mful_v, tmp_v, comb_v, stage_sh):
  c = lax.axis_index("c")
  s = lax.axis_index("s")
  w = c * NS + s
  zero = jnp.zeros((LN,), jnp.float32)

  pltpu.sync_copy(mpart_hbm.at[pl.ds(0, SPAD)], mful_v)
  pltpu.sync_copy(mpart_hbm.at[pl.ds(SPAD, SPAD)], tmp_v)

  def initm(i, _):
    sl = pl.ds(i * LN, LN)
    mful_v[sl] = jnp.maximum(mful_v[sl], tmp_v[sl])
    stat_v[sl] = zero
    return 0
  lax.fori_loop(0, SPAD // LN, initm, 0)

  pltpu.sync_copy(score_hbm.at[pl.ds(w * Q, Q)], sco_v)
  pltpu.sync_copy(batch_hbm.at[pl.ds(w * Q, Q)], ids_v)
  lane = lax.iota(jnp.int32, LN)

  def body(g, _):
    ids = ids_v[pl.ds(g * LN, LN)]
    sv = sco_v[pl.ds(g * LN, LN)]
    m = plsc.load_gather(mful_v, [ids])
    v = jnp.exp(sv - m)
    v = _seg_scan(ids, v, lane, lambda a, b, ok: a + jnp.where(ok, b, 0.0))
    last = _last_of_run(ids, lane)
    plsc.addupdate_scatter(stat_v, [ids], jnp.where(last, v, 0.0), mask=last)
    return 0
  lax.fori_loop(0, GRP, body, 0)

  pltpu.sync_copy(stat_v, stage_sh.at[s])
  plsc.subcore_barrier()
  pltpu.sync_copy(stage_sh.at[:, pl.ds(s * SEG, SEG)], comb_v)

  def comb(i, _):
    acc = comb_v[0, pl.ds(i * LN, LN)]
    for r in range(1, NS):
      acc = acc + comb_v[r, pl.ds(i * LN, LN)]
    stat_v[pl.ds(i * LN, LN)] = acc
    return 0
  lax.fori_loop(0, SEG // LN, comb, 0)
  pltpu.sync_copy(stat_v.at[pl.ds(0, SEG)],
                  dpart_hbm.at[pl.ds(c * SPAD + s * SEG, SEG)])


_denom = functools.partial(
    pl.kernel,
    out_type=jax.ShapeDtypeStruct((NC * SPAD,), jnp.float32),
    mesh=_mesh,
    compiler_params=_sc_params,
    scratch_types=[
        pltpu.VMEM((SPAD,), jnp.float32),
        pltpu.VMEM((Q,), jnp.float32),
        pltpu.VMEM((Q,), jnp.int32),
        pltpu.VMEM((SPAD,), jnp.float32),
        pltpu.VMEM((SPAD,), jnp.float32),
        pltpu.VMEM((NS, SEG), jnp.float32),
        pltpu.VMEM_SHARED((NS, SPAD), jnp.float32),
    ])(_denom_body)


NWIN = 5
MID = SPAD // NWIN
OSH = MID + LN
DUMP = MID
ZR = OSH // NS
DR = MID // NS
SSC = 80
NSS = RCH // SSC


def _pool_body(score_hbm, batch_hbm, f_hbm, mpart_hbm, dpart_hbm,
               sc_hbm, outp_hbm, mful_v, dful_v, tmp_v, ids_v, sco_v,
               fchunk_v, sidx_v, scbuf_v, out_sh):
  c = lax.axis_index("c")
  s = lax.axis_index("s")
  w = c * NS + s

  pltpu.sync_copy(mpart_hbm.at[pl.ds(0, SPAD)], mful_v)
  pltpu.sync_copy(dpart_hbm.at[pl.ds(0, SPAD)], dful_v)
  pltpu.sync_copy(batch_hbm.at[pl.ds(w * Q, Q)], ids_v)
  pltpu.sync_copy(score_hbm.at[pl.ds(w * Q, Q)], sco_v)
  pltpu.sync_copy(mpart_hbm.at[pl.ds(SPAD, SPAD)], tmp_v)

  def combm(i, _):
    sl = pl.ds(i * LN, LN)
    mful_v[sl] = jnp.maximum(mful_v[sl], tmp_v[sl])
    return 0
  lax.fori_loop(0, SPAD // LN, combm, 0)

  pltpu.sync_copy(dpart_hbm.at[pl.ds(SPAD, SPAD)], tmp_v)

  def combd(i, _):
    sl = pl.ds(i * LN, LN)
    dful_v[sl] = dful_v[sl] + tmp_v[sl]
    return 0
  lax.fori_loop(0, SPAD // LN, combd, 0)

  def cnt(g, acc):
    ids = ids_v[pl.ds(g * LN, LN)]
    return tuple(
        acc[k] + jnp.sum(jnp.where(ids < (k + 1) * MID, 1, 0)
                         .astype(jnp.int32))
        for k in range(NWIN - 1))
  pref = lax.fori_loop(0, GRP, cnt, (jnp.int32(0),) * (NWIN - 1))
  pref = (jnp.int32(0),) + pref + (jnp.int32(Q),)

  zero = jnp.zeros((LN,), jnp.float32)
  for win in range(NWIN):
    lo = win * MID
    def zbody(i, _):
      r = i // (D // LN)
      k = i % (D // LN)
      fchunk_v[r, pl.ds(k * LN, LN)] = zero
      return 0
    lax.fori_loop(0, ZR * (D // LN), zbody, 0)
    pltpu.sync_copy(fchunk_v.at[pl.ds(0, ZR)], out_sh.at[pl.ds(s * ZR, ZR)])
    plsc.subcore_barrier()

    jstart = pref[win] // RCH
    jend = (pref[win + 1] + RCH - 1) // RCH

    def chunk(j, _):
      base = w * Q + j * RCH
      pltpu.sync_copy(f_hbm.at[pl.ds(base, RCH)], fchunk_v)

      def grp(g, _):
        sl = pl.ds(j * RCH + g * LN, LN)
        ids = ids_v[sl]
        sv = sco_v[sl]
        m = plsc.load_gather(mful_v, [ids])
        dd = plsc.load_gather(dful_v, [ids])
        scv = jnp.exp(sv - m) / (dd + 1e-16)
        scbuf_v[pl.ds(j * RCH + g * LN, LN)] = scv
        tgt = jnp.where((ids >= lo) & (ids < lo + MID), ids - lo, DUMP)
        sidx_v[pl.ds(g * LN, LN)] = tgt
        for r in range(LN):
          srow = jnp.take_along_axis(
              scv, jnp.full((LN,), r, jnp.int32), axis=0,
              mode="promise_in_bounds")
          row = g * LN + r
          for k in range(D // LN):
            csl = pl.ds(k * LN, LN)
            fchunk_v[row, csl] = fchunk_v[row, csl] * srow
        return 0
      lax.fori_loop(0, RCH // LN, grp, 0)
      pltpu.sync_copy(fchunk_v, out_sh.at[sidx_v], add=True)
      return 0
    lax.fori_loop(jstart, jend, chunk, 0)

    plsc.subcore_barrier()
    pltpu.sync_copy(out_sh.at[pl.ds(s * DR, DR)], fchunk_v.at[pl.ds(0, DR)])
    pltpu.sync_copy(fchunk_v.at[pl.ds(0, DR)],
                    outp_hbm.at[pl.ds(c * SPAD + lo + s * DR, DR)])
    plsc.subcore_barrier()

  pltpu.sync_copy(scbuf_v, sc_hbm.at[pl.ds(w * Q, Q)])


_pool = functools.partial(
    pl.kernel,
    out_type=(jax.ShapeDtypeStruct((N,), jnp.float32),
              jax.ShapeDtypeStruct((NC * SPAD, D), jnp.float32)),
    mesh=_mesh,
    compiler_params=_sc_params,
    scratch_types=[
        pltpu.VMEM((SPAD,), jnp.float32),
        pltpu.VMEM((SPAD,), jnp.float32),
        pltpu.VMEM((SPAD,), jnp.float32),
        pltpu.VMEM((Q,), jnp.int32),
        pltpu.VMEM((Q,), jnp.float32),
        pltpu.VMEM((RCH, D), jnp.float32),
        pltpu.VMEM((RCH,), jnp.int32),
        pltpu.VMEM((Q,), jnp.float32),
        pltpu.VMEM_SHARED((OSH, D), jnp.float32),
    ])(_pool_body)


_BS = 1000


def _add_body(a_ref, b_ref, o_ref):
  o_ref[...] = a_ref[...] + b_ref[...]


def _combine_out(o0, o1):
  return pl.pallas_call(
      _add_body,
      grid=(S // _BS,),
      in_specs=[
          pl.BlockSpec((_BS, D), lambda i: (i, 0)),
          pl.BlockSpec((_BS, D), lambda i: (i, 0)),
      ],
      out_specs=pl.BlockSpec((_BS, D), lambda i: (i, 0)),
      out_shape=jax.ShapeDtypeStruct((S, D), jnp.float32),
  )(o0, o1)


@jax.jit
def kernel(feature, batch, Wa, ba, Wb, bb, Wc, bc):
  batch = batch.astype(jnp.int32)
  f, score2 = _dense(feature, Wa.T, ba.reshape(1, L), Wb.T, bb.reshape(1, L),
                     Wc.T, bc.reshape(1, 1))
  score = score2.reshape(N)
  mpart = _segmax(score, batch)
  dpart = _denom(score, batch, mpart)
  sc, outp = _pool(score, batch, f, mpart, dpart)
  out = _combine_out(outp[:S], outp[SPAD:SPAD + S])
  return out, sc.reshape(N, 1), f


# speedup vs baseline: 1.0704x; 1.0704x over previous
import functools

import jax
import jax.numpy as jnp
from jax import lax
from jax.experimental import pallas as pl
from jax.experimental.pallas import tpu as pltpu
from jax.experimental.pallas import tpu_sc as plsc

N = 320000
D = 128
L = 128
S = 10000

NC = 2
NS = 16
LN = 16
NW = NC * NS
Q = N // NW
SPAD = 10240
SEG = SPAD // NS
RCH = 400
NCHK = Q // RCH
GRP = Q // LN
NEG = -1e30

_mesh = plsc.VectorSubcoreMesh(
    core_axis_name="c", subcore_axis_name="s", num_cores=NC, num_subcores=NS)
_sc_params = pltpu.CompilerParams(needs_layout_passes=False)


_BR = 1600


def _dense_body(feat_ref, wat_ref, ba_ref, wbt_ref, bb_ref, wct_ref, bc_ref,
                f_ref, score_ref):
  x = feat_ref[...]
  ss = jnp.sum(x * x, axis=1, keepdims=True)
  f = x / jnp.maximum(jnp.sqrt(ss), 1e-12)
  fb = f.astype(jnp.bfloat16)
  a = jax.nn.sigmoid(
      jnp.dot(fb, wat_ref[...].astype(jnp.bfloat16),
              preferred_element_type=jnp.float32) + ba_ref[...])
  b = jnp.tanh(
      jnp.dot(fb, wbt_ref[...].astype(jnp.bfloat16),
              preferred_element_type=jnp.float32) + bb_ref[...])
  ab = a * b
  score = (jnp.dot(ab, wct_ref[...], preferred_element_type=jnp.float32)
           + bc_ref[0, 0])
  f_ref[...] = f
  score_ref[...] = score


def _dense(feature, WaT, ba2, WbT, bb2, WcT, bc2):
  return pl.pallas_call(
      _dense_body,
      grid=(N // _BR,),
      in_specs=[
          pl.BlockSpec((_BR, D), lambda i: (i, 0)),
          pl.BlockSpec((D, L), lambda i: (0, 0)),
          pl.BlockSpec((1, L), lambda i: (0, 0)),
          pl.BlockSpec((D, L), lambda i: (0, 0)),
          pl.BlockSpec((1, L), lambda i: (0, 0)),
          pl.BlockSpec((L, 1), lambda i: (0, 0)),
          pl.BlockSpec((1, 1), lambda i: (0, 0), memory_space=pltpu.SMEM),
      ],
      out_specs=[
          pl.BlockSpec((_BR, D), lambda i: (i, 0)),
          pl.BlockSpec((_BR, 1), lambda i: (i, 0)),
      ],
      out_shape=[
          jax.ShapeDtypeStruct((N, D), jnp.float32),
          jax.ShapeDtypeStruct((N, 1), jnp.float32),
      ],
  )(feature, WaT, ba2, WbT, bb2, WcT, bc2)


def _seg_scan(ids, v, lane, combine):
  for dlt in (1, 2, 4, 8):
    idx = jnp.maximum(lane - dlt, 0)
    vs = jnp.take_along_axis(v, idx, axis=0, mode="promise_in_bounds")
    is_ = jnp.take_along_axis(ids, idx, axis=0, mode="promise_in_bounds")
    ok = (lane >= dlt) & (is_ == ids)
    v = combine(v, vs, ok)
  return v


def _last_of_run(ids, lane):
  nxt = jnp.take_along_axis(ids, jnp.minimum(lane + 1, LN - 1), axis=0,
                            mode="promise_in_bounds")
  return (ids != nxt) | (lane == LN - 1)


def _segmax_body(score_hbm, batch_hbm, mpart_hbm, stat_v, sco_v, ids_v,
                 comb_v, stage_sh):
  c = lax.axis_index("c")
  s = lax.axis_index("s")
  w = c * NS + s
  neg = jnp.full((LN,), NEG, jnp.float32)

  def init(i, _):
    stat_v[pl.ds(i * LN, LN)] = neg
    return 0
  lax.fori_loop(0, SPAD // LN, init, 0)

  pltpu.sync_copy(score_hbm.at[pl.ds(w * Q, Q)], sco_v)
  pltpu.sync_copy(batch_hbm.at[pl.ds(w * Q, Q)], ids_v)
  lane = lax.iota(jnp.int32, LN)

  def body(g, _):
    ids = ids_v[pl.ds(g * LN, LN)]
    v = sco_v[pl.ds(g * LN, LN)]
    v = _seg_scan(ids, v, lane,
                  lambda a, b, ok: jnp.where(ok, jnp.maximum(a, b), a))
    last = _last_of_run(ids, lane)
    cur = plsc.load_gather(stat_v, [ids], mask=last)
    plsc.store_scatter(stat_v, [ids], jnp.maximum(cur, v), mask=last)
    return 0
  lax.fori_loop(0, GRP, body, 0)

  pltpu.sync_copy(stat_v, stage_sh.at[s])
  plsc.subcore_barrier()
  pltpu.sync_copy(stage_sh.at[:, pl.ds(s * SEG, SEG)], comb_v)

  def comb(i, _):
    acc = comb_v[0, pl.ds(i * LN, LN)]
    for r in range(1, NS):
      acc = jnp.maximum(acc, comb_v[r, pl.ds(i * LN, LN)])
    stat_v[pl.ds(i * LN, LN)] = acc
    return 0
  lax.fori_loop(0, SEG // LN, comb, 0)
  pltpu.sync_copy(stat_v.at[pl.ds(0, SEG)],
                  mpart_hbm.at[pl.ds(c * SPAD + s * SEG, SEG)])


_segmax = functools.partial(
    pl.kernel,
    out_type=jax.ShapeDtypeStruct((NC * SPAD,), jnp.float32),
    mesh=_mesh,
    compiler_params=_sc_params,
    scratch_types=[
        pltpu.VMEM((SPAD,), jnp.float32),
        pltpu.VMEM((Q,), jnp.float32),
        pltpu.VMEM((Q,), jnp.int32),
        pltpu.VMEM((NS, SEG), jnp.float32),
        pltpu.VMEM_SHARED((NS, SPAD), jnp.float32),
    ])(_segmax_body)


def _denom_body(score_hbm, batch_hbm, mpart_hbm, dpart_hbm, stat_v, sco_v,
                ids_v, mful_v, tmp_v, comb_v, stage_sh):
  c = lax.axis_index("c")
  s = lax.axis_index("s")
  w = c * NS + s
  zero = jnp.zeros((LN,), jnp.float32)

  pltpu.sync_copy(mpart_hbm.at[pl.ds(0, SPAD)], mful_v)
  pltpu.sync_copy(mpart_hbm.at[pl.ds(SPAD, SPAD)], tmp_v)

  def initm(i, _):
    sl = pl.ds(i * LN, LN)
    mful_v[sl] = jnp.maximum(mful_v[sl], tmp_v[sl])
    stat_v[sl] = zero
    return 0
  lax.fori_loop(0, SPAD // LN, initm, 0)

  pltpu.sync_copy(score_hbm.at[pl.ds(w * Q, Q)], sco_v)
  pltpu.sync_copy(batch_hbm.at[pl.ds(w * Q, Q)], ids_v)
  lane = lax.iota(jnp.int32, LN)

  def body(g, _):
    ids = ids_v[pl.ds(g * LN, LN)]
    sv = sco_v[pl.ds(g * LN, LN)]
    m = plsc.load_gather(mful_v, [ids])
    v = jnp.exp(sv - m)
    v = _seg_scan(ids, v, lane, lambda a, b, ok: a + jnp.where(ok, b, 0.0))
    last = _last_of_run(ids, lane)
    plsc.addupdate_scatter(stat_v, [ids], jnp.where(last, v, 0.0), mask=last)
    return 0
  lax.fori_loop(0, GRP, body, 0)

  pltpu.sync_copy(stat_v, stage_sh.at[s])
  plsc.subcore_barrier()
  pltpu.sync_copy(stage_sh.at[:, pl.ds(s * SEG, SEG)], comb_v)

  def comb(i, _):
    acc = comb_v[0, pl.ds(i * LN, LN)]
    for r in range(1, NS):
      acc = acc + comb_v[r, pl.ds(i * LN, LN)]
    stat_v[pl.ds(i * LN, LN)] = acc
    return 0
  lax.fori_loop(0, SEG // LN, comb, 0)
  pltpu.sync_copy(stat_v.at[pl.ds(0, SEG)],
                  dpart_hbm.at[pl.ds(c * SPAD + s * SEG, SEG)])


_denom = functools.partial(
    pl.kernel,
    out_type=jax.ShapeDtypeStruct((NC * SPAD,), jnp.float32),
    mesh=_mesh,
    compiler_params=_sc_params,
    scratch_types=[
        pltpu.VMEM((SPAD,), jnp.float32),
        pltpu.VMEM((Q,), jnp.float32),
        pltpu.VMEM((Q,), jnp.int32),
        pltpu.VMEM((SPAD,), jnp.float32),
        pltpu.VMEM((SPAD,), jnp.float32),
        pltpu.VMEM((NS, SEG), jnp.float32),
        pltpu.VMEM_SHARED((NS, SPAD), jnp.float32),
    ])(_denom_body)


NWIN = 4
MID = SPAD // NWIN
OSH = MID + LN
DUMP = MID
ZR = OSH // NS
DR = MID // NS
SSC = 80
NSS = RCH // SSC


def _pool_body(score_hbm, batch_hbm, f_hbm, mpart_hbm, dpart_hbm,
               sc_hbm, outp_hbm, mful_v, dful_v, tmp_v, ids_v, sco_v,
               fchunk_v, sidx_v, scbuf_v, out_sh):
  c = lax.axis_index("c")
  s = lax.axis_index("s")
  w = c * NS + s

  pltpu.sync_copy(mpart_hbm.at[pl.ds(0, SPAD)], mful_v)
  pltpu.sync_copy(dpart_hbm.at[pl.ds(0, SPAD)], dful_v)
  pltpu.sync_copy(batch_hbm.at[pl.ds(w * Q, Q)], ids_v)
  pltpu.sync_copy(score_hbm.at[pl.ds(w * Q, Q)], sco_v)
  pltpu.sync_copy(mpart_hbm.at[pl.ds(SPAD, SPAD)], tmp_v)

  def combm(i, _):
    sl = pl.ds(i * LN, LN)
    mful_v[sl] = jnp.maximum(mful_v[sl], tmp_v[sl])
    return 0
  lax.fori_loop(0, SPAD // LN, combm, 0)

  pltpu.sync_copy(dpart_hbm.at[pl.ds(SPAD, SPAD)], tmp_v)

  def combd(i, _):
    sl = pl.ds(i * LN, LN)
    dful_v[sl] = dful_v[sl] + tmp_v[sl]
    return 0
  lax.fori_loop(0, SPAD // LN, combd, 0)

  def cnt(g, acc):
    ids = ids_v[pl.ds(g * LN, LN)]
    return tuple(
        acc[k] + jnp.sum(jnp.where(ids < (k + 1) * MID, 1, 0)
                         .astype(jnp.int32))
        for k in range(NWIN - 1))
  pref = lax.fori_loop(0, GRP, cnt, (jnp.int32(0),) * (NWIN - 1))
  pref = (jnp.int32(0),) + pref + (jnp.int32(Q),)

  zero = jnp.zeros((LN,), jnp.float32)
  for win in range(NWIN):
    lo = win * MID
    def zbody(i, _):
      r = i // (D // LN)
      k = i % (D // LN)
      fchunk_v[r, pl.ds(k * LN, LN)] = zero
      return 0
    lax.fori_loop(0, ZR * (D // LN), zbody, 0)
    pltpu.sync_copy(fchunk_v.at[pl.ds(0, ZR)], out_sh.at[pl.ds(s * ZR, ZR)])
    plsc.subcore_barrier()

    jstart = pref[win] // RCH
    jend = (pref[win + 1] + RCH - 1) // RCH

    def chunk(j, _):
      base = w * Q + j * RCH
      pltpu.sync_copy(f_hbm.at[pl.ds(base, RCH)], fchunk_v)

      for j5 in range(NSS):
        def grp(gg, _):
          g = j5 * (SSC // LN) + gg
          sl = pl.ds(j * RCH + g * LN, LN)
          ids = ids_v[sl]
          sv = sco_v[sl]
          m = plsc.load_gather(mful_v, [ids])
          dd = plsc.load_gather(dful_v, [ids])
          scv = jnp.exp(sv - m) / (dd + 1e-16)
          scbuf_v[pl.ds(g * LN, LN)] = scv
          tgt = jnp.where((ids >= lo) & (ids < lo + MID), ids - lo, DUMP)
          sidx_v[pl.ds(gg * LN, LN)] = tgt
          for r in range(LN):
            srow = jnp.take_along_axis(
                scv, jnp.full((LN,), r, jnp.int32), axis=0,
                mode="promise_in_bounds")
            row = g * LN + r
            for k in range(D // LN):
              csl = pl.ds(k * LN, LN)
              fchunk_v[row, csl] = fchunk_v[row, csl] * srow
          return 0
        lax.fori_loop(0, SSC // LN, grp, 0)
        pltpu.sync_copy(fchunk_v.at[pl.ds(j5 * SSC, SSC)],
                        out_sh.at[sidx_v], add=True)

      pltpu.sync_copy(scbuf_v, sc_hbm.at[pl.ds(base, RCH)])
      return 0
    lax.fori_loop(jstart, jend, chunk, 0)

    plsc.subcore_barrier()
    pltpu.sync_copy(out_sh.at[pl.ds(s * DR, DR)], fchunk_v.at[pl.ds(0, DR)])
    pltpu.sync_copy(fchunk_v.at[pl.ds(0, DR)],
                    outp_hbm.at[pl.ds(c * SPAD + lo + s * DR, DR)])
    plsc.subcore_barrier()


_pool = functools.partial(
    pl.kernel,
    out_type=(jax.ShapeDtypeStruct((N,), jnp.float32),
              jax.ShapeDtypeStruct((NC * SPAD, D), jnp.float32)),
    mesh=_mesh,
    compiler_params=_sc_params,
    scratch_types=[
        pltpu.VMEM((SPAD,), jnp.float32),
        pltpu.VMEM((SPAD,), jnp.float32),
        pltpu.VMEM((SPAD,), jnp.float32),
        pltpu.VMEM((Q,), jnp.int32),
        pltpu.VMEM((Q,), jnp.float32),
        pltpu.VMEM((RCH, D), jnp.float32),
        pltpu.VMEM((SSC,), jnp.int32),
        pltpu.VMEM((RCH,), jnp.float32),
        pltpu.VMEM_SHARED((OSH, D), jnp.float32),
    ])(_pool_body)


_BS = 1000


def _add_body(a_ref, b_ref, o_ref):
  o_ref[...] = a_ref[...] + b_ref[...]


def _combine_out(o0, o1):
  return pl.pallas_call(
      _add_body,
      grid=(S // _BS,),
      in_specs=[
          pl.BlockSpec((_BS, D), lambda i: (i, 0)),
          pl.BlockSpec((_BS, D), lambda i: (i, 0)),
      ],
      out_specs=pl.BlockSpec((_BS, D), lambda i: (i, 0)),
      out_shape=jax.ShapeDtypeStruct((S, D), jnp.float32),
  )(o0, o1)


@jax.jit
def kernel(feature, batch, Wa, ba, Wb, bb, Wc, bc):
  batch = batch.astype(jnp.int32)
  f, score2 = _dense(feature, Wa.T, ba.reshape(1, L), Wb.T, bb.reshape(1, L),
                     Wc.T, bc.reshape(1, 1))
  score = score2.reshape(N)
  mpart = _segmax(score, batch)
  dpart = _denom(score, batch, mpart)
  sc, outp = _pool(score, batch, f, mpart, dpart)
  out = _combine_out(outp[:S], outp[SPAD:SPAD + S])
  return out, sc.reshape(N, 1), f


# speedup vs baseline: 1.0825x; 1.0113x over previous
import functools

import jax
import jax.numpy as jnp
from jax import lax
from jax.experimental import pallas as pl
from jax.experimental.pallas import tpu as pltpu
from jax.experimental.pallas import tpu_sc as plsc

N = 320000
D = 128
L = 128
S = 10000

NC = 2
NS = 16
LN = 16
NW = NC * NS
Q = N // NW
SPAD = 10240
SEG = SPAD // NS
RCH = 400
NCHK = Q // RCH
GRP = Q // LN
NEG = -1e30

_mesh = plsc.VectorSubcoreMesh(
    core_axis_name="c", subcore_axis_name="s", num_cores=NC, num_subcores=NS)
_sc_params = pltpu.CompilerParams(needs_layout_passes=False)


_BR = 1600


def _dense_body(feat_ref, wat_ref, ba_ref, wbt_ref, bb_ref, wct_ref, bc_ref,
                f_ref, score_ref):
  x = feat_ref[...]
  ss = jnp.sum(x * x, axis=1, keepdims=True)
  f = x / jnp.maximum(jnp.sqrt(ss), 1e-12)
  a = jax.nn.sigmoid(
      jnp.dot(f, wat_ref[...], preferred_element_type=jnp.float32)
      + ba_ref[...])
  b = jnp.tanh(
      jnp.dot(f, wbt_ref[...], preferred_element_type=jnp.float32)
      + bb_ref[...])
  ab = a * b
  score = (jnp.dot(ab, wct_ref[...], preferred_element_type=jnp.float32)
           + bc_ref[0, 0])
  f_ref[...] = f
  score_ref[...] = score


def _dense(feature, WaT, ba2, WbT, bb2, WcT, bc2):
  return pl.pallas_call(
      _dense_body,
      grid=(N // _BR,),
      in_specs=[
          pl.BlockSpec((_BR, D), lambda i: (i, 0)),
          pl.BlockSpec((D, L), lambda i: (0, 0)),
          pl.BlockSpec((1, L), lambda i: (0, 0)),
          pl.BlockSpec((D, L), lambda i: (0, 0)),
          pl.BlockSpec((1, L), lambda i: (0, 0)),
          pl.BlockSpec((L, 1), lambda i: (0, 0)),
          pl.BlockSpec((1, 1), lambda i: (0, 0), memory_space=pltpu.SMEM),
      ],
      out_specs=[
          pl.BlockSpec((_BR, D), lambda i: (i, 0)),
          pl.BlockSpec((_BR, 1), lambda i: (i, 0)),
      ],
      out_shape=[
          jax.ShapeDtypeStruct((N, D), jnp.float32),
          jax.ShapeDtypeStruct((N, 1), jnp.float32),
      ],
  )(feature, WaT, ba2, WbT, bb2, WcT, bc2)


def _seg_scan(ids, v, lane, combine):
  for dlt in (1, 2, 4, 8):
    idx = jnp.maximum(lane - dlt, 0)
    vs = jnp.take_along_axis(v, idx, axis=0, mode="promise_in_bounds")
    is_ = jnp.take_along_axis(ids, idx, axis=0, mode="promise_in_bounds")
    ok = (lane >= dlt) & (is_ == ids)
    v = combine(v, vs, ok)
  return v


def _seg_scan_rev(ids, v, lane, combine):
  for dlt in (1, 2, 4, 8):
    idx = jnp.minimum(lane + dlt, LN - 1)
    vs = jnp.take_along_axis(v, idx, axis=0, mode="promise_in_bounds")
    is_ = jnp.take_along_axis(ids, idx, axis=0, mode="promise_in_bounds")
    ok = (lane + dlt <= LN - 1) & (is_ == ids)
    v = combine(v, vs, ok)
  return v


def _last_of_run(ids, lane):
  nxt = jnp.take_along_axis(ids, jnp.minimum(lane + 1, LN - 1), axis=0,
                            mode="promise_in_bounds")
  return (ids != nxt) | (lane == LN - 1)


def _stats_body(score_hbm, batch_hbm, mpart_hbm, dpart_hbm, mstat_v, dstat_v,
                sco_v, ids_v, comb_m, comb_d, stage_m, stage_d):
  c = lax.axis_index("c")
  s = lax.axis_index("s")
  w = c * NS + s
  neg = jnp.full((LN,), NEG, jnp.float32)
  zero = jnp.zeros((LN,), jnp.float32)

  def init(i, _):
    mstat_v[pl.ds(i * LN, LN)] = neg
    dstat_v[pl.ds(i * LN, LN)] = zero
    return 0
  lax.fori_loop(0, SPAD // LN, init, 0)

  pltpu.sync_copy(score_hbm.at[pl.ds(w * Q, Q)], sco_v)
  pltpu.sync_copy(batch_hbm.at[pl.ds(w * Q, Q)], ids_v)
  lane = lax.iota(jnp.int32, LN)

  def body(g, _):
    ids = ids_v[pl.ds(g * LN, LN)]
    sv = sco_v[pl.ds(g * LN, LN)]
    rmax = _seg_scan(ids, sv, lane,
                     lambda a, b, ok: jnp.where(ok, jnp.maximum(a, b), a))
    rmax = _seg_scan_rev(ids, rmax, lane,
                         lambda a, b, ok: jnp.where(ok, jnp.maximum(a, b), a))
    p = jnp.exp(sv - rmax)
    psum = _seg_scan(ids, p, lane, lambda a, b, ok: a + jnp.where(ok, b, 0.0))
    last = _last_of_run(ids, lane)
    mo = plsc.load_gather(mstat_v, [ids], mask=last)
    do = plsc.load_gather(dstat_v, [ids], mask=last)
    mn = jnp.maximum(mo, rmax)
    dn = do * jnp.exp(mo - mn) + psum * jnp.exp(rmax - mn)
    plsc.store_scatter(mstat_v, [ids], mn, mask=last)
    plsc.store_scatter(dstat_v, [ids], dn, mask=last)
    return 0
  lax.fori_loop(0, GRP, body, 0)

  pltpu.sync_copy(mstat_v, stage_m.at[s])
  pltpu.sync_copy(dstat_v, stage_d.at[s])
  plsc.subcore_barrier()
  pltpu.sync_copy(stage_m.at[:, pl.ds(s * SEG, SEG)], comb_m)
  pltpu.sync_copy(stage_d.at[:, pl.ds(s * SEG, SEG)], comb_d)

  def comb(i, _):
    sl = pl.ds(i * LN, LN)
    mm = comb_m[0, sl]
    for r in range(1, NS):
      mm = jnp.maximum(mm, comb_m[r, sl])
    dd = comb_d[0, sl] * jnp.exp(comb_m[0, sl] - mm)
    for r in range(1, NS):
      dd = dd + comb_d[r, sl] * jnp.exp(comb_m[r, sl] - mm)
    mstat_v[sl] = mm
    dstat_v[sl] = dd
    return 0
  lax.fori_loop(0, SEG // LN, comb, 0)
  pltpu.sync_copy(mstat_v.at[pl.ds(0, SEG)],
                  mpart_hbm.at[pl.ds(c * SPAD + s * SEG, SEG)])
  pltpu.sync_copy(dstat_v.at[pl.ds(0, SEG)],
                  dpart_hbm.at[pl.ds(c * SPAD + s * SEG, SEG)])


_stats = functools.partial(
    pl.kernel,
    out_type=(jax.ShapeDtypeStruct((NC * SPAD,), jnp.float32),
              jax.ShapeDtypeStruct((NC * SPAD,), jnp.float32)),
    mesh=_mesh,
    compiler_params=_sc_params,
    scratch_types=[
        pltpu.VMEM((SPAD,), jnp.float32),
        pltpu.VMEM((SPAD,), jnp.float32),
        pltpu.VMEM((Q,), jnp.float32),
        pltpu.VMEM((Q,), jnp.int32),
        pltpu.VMEM((NS, SEG), jnp.float32),
        pltpu.VMEM((NS, SEG), jnp.float32),
        pltpu.VMEM_SHARED((NS, SPAD), jnp.float32),
        pltpu.VMEM_SHARED((NS, SPAD), jnp.float32),
    ])(_stats_body)


NWIN = 4
MID = SPAD // NWIN
OSH = MID + LN
DUMP = MID
ZR = OSH // NS
DR = MID // NS
SSC = 80
NSS = RCH // SSC


def _pool_body(score_hbm, batch_hbm, f_hbm, mpart_hbm, dpart_hbm,
               sc_hbm, outp_hbm, mful_v, dful_v, tmp_v, ids_v, sco_v,
               fchunk_v, sidx_v, scbuf_v, out_sh):
  c = lax.axis_index("c")
  s = lax.axis_index("s")
  w = c * NS + s

  pltpu.sync_copy(mpart_hbm.at[pl.ds(0, SPAD)], mful_v)
  pltpu.sync_copy(dpart_hbm.at[pl.ds(0, SPAD)], dful_v)
  pltpu.sync_copy(batch_hbm.at[pl.ds(w * Q, Q)], ids_v)
  pltpu.sync_copy(mpart_hbm.at[pl.ds(SPAD, SPAD)], tmp_v)
  pltpu.sync_copy(dpart_hbm.at[pl.ds(SPAD, SPAD)], sco_v)

  def combm(i, _):
    sl = pl.ds(i * LN, LN)
    m1 = tmp_v[sl]
    mm = jnp.maximum(mful_v[sl], m1)
    dful_v[sl] = (dful_v[sl] * jnp.exp(mful_v[sl] - mm)
                  + sco_v[sl] * jnp.exp(m1 - mm))
    mful_v[sl] = mm
    return 0
  lax.fori_loop(0, SPAD // LN, combm, 0)

  pltpu.sync_copy(score_hbm.at[pl.ds(w * Q, Q)], sco_v.at[pl.ds(0, Q)])

  def cnt(g, acc):
    ids = ids_v[pl.ds(g * LN, LN)]
    return tuple(
        acc[k] + jnp.sum(jnp.where(ids < (k + 1) * MID, 1, 0)
                         .astype(jnp.int32))
        for k in range(NWIN - 1))
  pref = lax.fori_loop(0, GRP, cnt, (jnp.int32(0),) * (NWIN - 1))
  pref = (jnp.int32(0),) + pref + (jnp.int32(Q),)

  zero = jnp.zeros((LN,), jnp.float32)
  for win in range(NWIN):
    lo = win * MID
    def zbody(i, _):
      r = i // (D // LN)
      k = i % (D // LN)
      fchunk_v[r, pl.ds(k * LN, LN)] = zero
      return 0
    lax.fori_loop(0, ZR * (D // LN), zbody, 0)
    pltpu.sync_copy(fchunk_v.at[pl.ds(0, ZR)], out_sh.at[pl.ds(s * ZR, ZR)])
    plsc.subcore_barrier()

    jstart = pref[win] // RCH
    jend = (pref[win + 1] + RCH - 1) // RCH

    def chunk(j, _):
      base = w * Q + j * RCH
      pltpu.sync_copy(f_hbm.at[pl.ds(base, RCH)], fchunk_v)

      for j5 in range(NSS):
        def grp(gg, _):
          g = j5 * (SSC // LN) + gg
          sl = pl.ds(j * RCH + g * LN, LN)
          ids = ids_v[sl]
          sv = sco_v[sl]
          m = plsc.load_gather(mful_v, [ids])
          dd = plsc.load_gather(dful_v, [ids])
          scv = jnp.exp(sv - m) / (dd + 1e-16)
          scbuf_v[pl.ds(g * LN, LN)] = scv
          tgt = jnp.where((ids >= lo) & (ids < lo + MID), ids - lo, DUMP)
          sidx_v[pl.ds(gg * LN, LN)] = tgt
          for r in range(LN):
            srow = jnp.take_along_axis(
                scv, jnp.full((LN,), r, jnp.int32), axis=0,
                mode="promise_in_bounds")
            row = g * LN + r
            for k in range(D // LN):
              csl = pl.ds(k * LN, LN)
              fchunk_v[row, csl] = fchunk_v[row, csl] * srow
          return 0
        lax.fori_loop(0, SSC // LN, grp, 0)
        pltpu.sync_copy(fchunk_v.at[pl.ds(j5 * SSC, SSC)],
                        out_sh.at[sidx_v], add=True)

      pltpu.sync_copy(scbuf_v, sc_hbm.at[pl.ds(base, RCH)])
      return 0
    lax.fori_loop(jstart, jend, chunk, 0)

    plsc.subcore_barrier()
    pltpu.sync_copy(out_sh.at[pl.ds(s * DR, DR)], fchunk_v.at[pl.ds(0, DR)])
    pltpu.sync_copy(fchunk_v.at[pl.ds(0, DR)],
                    outp_hbm.at[pl.ds(c * SPAD + lo + s * DR, DR)])
    plsc.subcore_barrier()


_pool = functools.partial(
    pl.kernel,
    out_type=(jax.ShapeDtypeStruct((N,), jnp.float32),
              jax.ShapeDtypeStruct((NC * SPAD, D), jnp.float32)),
    mesh=_mesh,
    compiler_params=_sc_params,
    scratch_types=[
        pltpu.VMEM((SPAD,), jnp.float32),
        pltpu.VMEM((SPAD,), jnp.float32),
        pltpu.VMEM((SPAD,), jnp.float32),
        pltpu.VMEM((Q,), jnp.int32),
        pltpu.VMEM((SPAD,), jnp.float32),
        pltpu.VMEM((RCH, D), jnp.float32),
        pltpu.VMEM((SSC,), jnp.int32),
        pltpu.VMEM((RCH,), jnp.float32),
        pltpu.VMEM_SHARED((OSH, D), jnp.float32),
    ])(_pool_body)


_BS = 1000


def _add_body(a_ref, b_ref, o_ref):
  o_ref[...] = a_ref[...] + b_ref[...]


def _combine_out(o0, o1):
  return pl.pallas_call(
      _add_body,
      grid=(S // _BS,),
      in_specs=[
          pl.BlockSpec((_BS, D), lambda i: (i, 0)),
          pl.BlockSpec((_BS, D), lambda i: (i, 0)),
      ],
      out_specs=pl.BlockSpec((_BS, D), lambda i: (i, 0)),
      out_shape=jax.ShapeDtypeStruct((S, D), jnp.float32),
  )(o0, o1)


@jax.jit
def kernel(feature, batch, Wa, ba, Wb, bb, Wc, bc):
  batch = batch.astype(jnp.int32)
  f, score2 = _dense(feature, Wa.T, ba.reshape(1, L), Wb.T, bb.reshape(1, L),
                     Wc.T, bc.reshape(1, 1))
  score = score2.reshape(N)
  mpart, dpart = _stats(score, batch)
  sc, outp = _pool(score, batch, f, mpart, dpart)
  out = _combine_out(outp[:S], outp[SPAD:SPAD + S])
  return out, sc.reshape(N, 1), f
